# Initial kernel scaffold; baseline (speedup 1.0000x reference)
import jax
import jax.numpy as jnp
from jax.experimental import pallas as pl


def kernel(p, x, o, Wq, bq, Wk, bk, Wv, bv, Wp1, bp1, gamma, beta, Wp2, bp2):
    raise NotImplementedError("write your pallas kernel here")


# speedup vs baseline: 3.3243x; 3.3243x over previous
import functools
import math

import jax
import jax.numpy as jnp
from jax import lax
from jax.experimental import pallas as pl
from jax.experimental.pallas import tpu as pltpu
from jax.experimental.pallas import tpu_sc as plsc

N = 8192
C = 256
NS = 16


_QKV_R = 512


def _qkv_body(x_ref, wq_ref, bq_ref, wk_ref, bk_ref, wv_ref, bv_ref,
              q_ref, k_ref, v_ref):
    xb = x_ref[...]
    q_ref[...] = jnp.dot(xb, wq_ref[...],
                         preferred_element_type=jnp.float32) + bq_ref[...]
    k_ref[...] = jnp.dot(xb, wk_ref[...],
                         preferred_element_type=jnp.float32) + bk_ref[...]
    v_ref[...] = jnp.dot(xb, wv_ref[...],
                         preferred_element_type=jnp.float32) + bv_ref[...]


def _qkv(x, Wq, bq, Wk, bk, Wv, bv):
    grid = (N // _QKV_R,)
    row_spec = pl.BlockSpec((_QKV_R, C), lambda i: (i, 0))
    w_spec = pl.BlockSpec((C, C), lambda i: (0, 0))
    b_spec = pl.BlockSpec((1, C), lambda i: (0, 0))
    return pl.pallas_call(
        _qkv_body,
        grid=grid,
        in_specs=[row_spec, w_spec, b_spec, w_spec, b_spec, w_spec, b_spec],
        out_specs=[row_spec, row_spec, row_spec],
        out_shape=[jax.ShapeDtypeStruct((N, C), jnp.float32)] * 3,
    )(x, Wq, bq.reshape(1, C), Wk, bk.reshape(1, C), Wv, bv.reshape(1, C))


_KNN_R = 128


def _knn_body(pb_ref, pt_ref, idx_ref):
    pb = pb_ref[...]
    pt = pt_ref[...]
    sqb = jnp.sum(pb * pb, axis=1, keepdims=True)
    sqa = jnp.sum(pt * pt, axis=0, keepdims=True)
    d2 = sqb + sqa - 2.0 * jnp.dot(
        pb, pt, preferred_element_type=jnp.float32)
    iota = lax.broadcasted_iota(jnp.int32, d2.shape, 1)
    iota16 = lax.broadcasted_iota(jnp.int32, (_KNN_R, NS), 1)

    def step(t, carry):
        cur, acc = carry
        m = jnp.min(cur, axis=1, keepdims=True)
        cand = jnp.where(cur == m, iota, jnp.int32(2 ** 30))
        j = jnp.min(cand, axis=1, keepdims=True)
        cur = jnp.where(iota == j, jnp.float32(jnp.inf), cur)
        acc = jnp.where(iota16 == t, j, acc)
        return cur, acc

    _, acc = lax.fori_loop(
        0, NS, step, (d2, jnp.zeros((_KNN_R, NS), jnp.int32)))
    idx_ref[...] = acc


def _knn(p8, pT8):
    grid = (N // _KNN_R,)
    return pl.pallas_call(
        _knn_body,
        grid=grid,
        in_specs=[pl.BlockSpec((_KNN_R, 8), lambda i: (i, 0)),
                  pl.BlockSpec((8, N), lambda i: (0, 0))],
        out_specs=pl.BlockSpec((_KNN_R, NS), lambda i: (i, 0)),
        out_shape=jax.ShapeDtypeStruct((N, NS), jnp.int32),
    )(p8, pT8)


_G_CH = 128
_NW = 32
_B = N * NS
_PER_W = _B // _NW


def _gather_sc(idx_flat, xk, xv, p128):
    mesh = plsc.VectorSubcoreMesh(core_axis_name="c", subcore_axis_name="s")

    @functools.partial(
        pl.kernel,
        mesh=mesh,
        out_type=[
            jax.ShapeDtypeStruct((_B, C), jnp.float32),
            jax.ShapeDtypeStruct((_B, C), jnp.float32),
            jax.ShapeDtypeStruct((_B, 128), jnp.float32),
        ],
        scratch_types=[
            pltpu.VMEM((_G_CH,), jnp.int32),
            pltpu.VMEM((_G_CH, C), jnp.float32),
            pltpu.VMEM((_G_CH, C), jnp.float32),
            pltpu.VMEM((_G_CH, 128), jnp.float32),
            pltpu.SemaphoreType.DMA,
            pltpu.SemaphoreType.DMA,
            pltpu.SemaphoreType.DMA,
        ],
    )
    def k(idx_hbm, xk_hbm, xv_hbm, p128_hbm, gk_hbm, gv_hbm, gp_hbm,
          idx_v, bk, bv, bp, s1, s2, s3):
        wid = lax.axis_index("s") * 2 + lax.axis_index("c")

        def body(ch, carry):
            base = wid * _PER_W + ch * _G_CH
            pltpu.sync_copy(idx_hbm.at[pl.ds(base, _G_CH)], idx_v)
            c1 = pltpu.async_copy(xk_hbm.at[idx_v], bk, s1)
            c2 = pltpu.async_copy(xv_hbm.at[idx_v], bv, s2)
            c3 = pltpu.async_copy(p128_hbm.at[idx_v], bp, s3)
            c1.wait()
            c2.wait()
            c3.wait()
            pltpu.sync_copy(bk, gk_hbm.at[pl.ds(base, _G_CH)])
            pltpu.sync_copy(bv, gv_hbm.at[pl.ds(base, _G_CH)])
            pltpu.sync_copy(bp, gp_hbm.at[pl.ds(base, _G_CH)])
            return carry

        lax.fori_loop(0, _PER_W // _G_CH, body, 0)

    return k(idx_flat, xk, xv, p128)


_ST_R = 512


def _stats_body(gp_ref, p_ref, out_ref):
    pg = gp_ref[...]
    pb = p_ref[...]
    pr = pg[:, :, :3] - pb[:, :, :3]
    d0 = pr[:, :, 0:1]
    d1 = pr[:, :, 1:2]
    d2 = pr[:, :, 2:3]
    vals = [
        jnp.sum(d0), jnp.sum(d1), jnp.sum(d2),
        jnp.sum(d0 * d0), jnp.sum(d0 * d1), jnp.sum(d0 * d2),
        jnp.sum(d1 * d1), jnp.sum(d1 * d2), jnp.sum(d2 * d2),
    ]
    vec = jnp.concatenate(
        [v.reshape(1, 1) for v in vals] + [jnp.zeros((1, 7), jnp.float32)],
        axis=1)

    @pl.when(pl.program_id(0) == 0)
    def _():
        out_ref[...] = jnp.zeros_like(out_ref)

    out_ref[...] += vec


def _stats(gp3, p3):
    grid = (N // _ST_R,)
    return pl.pallas_call(
        _stats_body,
        grid=grid,
        in_specs=[pl.BlockSpec((_ST_R, NS, 128), lambda i: (i, 0, 0)),
                  pl.BlockSpec((_ST_R, 1, 128), lambda i: (i, 0, 0))],
        out_specs=pl.BlockSpec((1, 16), lambda i: (0, 0)),
        out_shape=jax.ShapeDtypeStruct((1, 16), jnp.float32),
    )(gp3, p3)


_AT_R = 128


def _attn_body(xq_ref, gk_ref, gv_ref, gp_ref, p_ref, stats_ref,
               wp1_ref, bp1_ref, gamma_ref, beta_ref, wp2_ref, bp2_ref,
               out_ref):
    pg = gp_ref[...]
    pb = p_ref[...]
    pr = pg[:, :, :3] - pb[:, :, :3]
    prd = [pr[:, :, 0:1], pr[:, :, 1:2], pr[:, :, 2:3]]

    M = float(N * NS)
    s1 = [stats_ref[0, i] for i in range(3)]
    s2 = {(0, 0): stats_ref[0, 3], (0, 1): stats_ref[0, 4],
          (0, 2): stats_ref[0, 5], (1, 1): stats_ref[0, 6],
          (1, 2): stats_ref[0, 7], (2, 2): stats_ref[0, 8]}

    pe = jnp.zeros(gk_ref.shape, jnp.float32) + wp2_ref[3, :].reshape(1, 1, C)
    for c in range(3):
        w = [wp1_ref[d, c] for d in range(3)]
        b = bp1_ref[c]
        h = prd[0] * w[0] + prd[1] * w[1] + prd[2] * w[2] + b
        sw = s1[0] * w[0] + s1[1] * w[1] + s1[2] * w[2]
        mean = sw / M + b
        ex2 = (w[0] * w[0] * s2[(0, 0)] + w[1] * w[1] * s2[(1, 1)]
               + w[2] * w[2] * s2[(2, 2)]
               + 2.0 * (w[0] * w[1] * s2[(0, 1)]
                        + w[0] * w[2] * s2[(0, 2)]
                        + w[1] * w[2] * s2[(1, 2)])) / M \
            + 2.0 * b * sw / M + b * b
        var = ex2 - mean * mean
        inv = gamma_ref[c] / jnp.sqrt(var + 1e-5)
        hn = (h - mean) * inv + beta_ref[c]
        r = jnp.maximum(hn, 0.0)
        pe = pe + r * wp2_ref[c, :].reshape(1, 1, C)

    gk = gk_ref[...] + pe
    gv = gv_ref[...] + pe
    xq = xq_ref[...]
    attn = jnp.sum(xq * gk, axis=2, keepdims=True) * (1.0 / 16.0)
    attn = attn - jnp.max(attn, axis=1, keepdims=True)
    e = jnp.exp(attn)
    wgt = e / jnp.sum(e, axis=1, keepdims=True)
    out_ref[...] = jnp.sum(gv * wgt, axis=1, keepdims=True)


def _attention(xq3, gk3, gv3, gp3, p3, stats, Wp1, bp1, gamma, beta,
               Wp2b, bp2):
    grid = (N // _AT_R,)
    smem = functools.partial(pl.BlockSpec, memory_space=pltpu.SMEM)
    out = pl.pallas_call(
        _attn_body,
        grid=grid,
        in_specs=[
            pl.BlockSpec((_AT_R, 1, C), lambda i: (i, 0, 0)),
            pl.BlockSpec((_AT_R, NS, C), lambda i: (i, 0, 0)),
            pl.BlockSpec((_AT_R, NS, C), lambda i: (i, 0, 0)),
            pl.BlockSpec((_AT_R, NS, 128), lambda i: (i, 0, 0)),
            pl.BlockSpec((_AT_R, 1, 128), lambda i: (i, 0, 0)),
            smem(),
            smem(),
            smem(),
            smem(),
            smem(),
            pl.BlockSpec((4, C), lambda i: (0, 0)),
            smem(),
        ],
        out_specs=pl.BlockSpec((_AT_R, 1, C), lambda i: (i, 0, 0)),
        out_shape=jax.ShapeDtypeStruct((N, 1, C), jnp.float32),
    )(xq3, gk3, gv3, gp3, p3, stats, Wp1, bp1, gamma, beta, Wp2b, bp2)
    return out.reshape(N, C)


def kernel(p, x, o, Wq, bq, Wk, bk, Wv, bv, Wp1, bp1, gamma, beta, Wp2,
           bp2):
    del o
    x_q, x_k, x_v = _qkv(x, Wq, bq, Wk, bk, Wv, bv)

    p8 = jnp.pad(p, ((0, 0), (0, 5)))
    pT8 = p8.T
    idx = _knn(p8, pT8)

    p128 = jnp.pad(p, ((0, 0), (0, 125)))
    gk, gv, gp = _gather_sc(idx.reshape(_B), x_k, x_v, p128)

    gp3 = gp.reshape(N, NS, 128)
    p3 = p128.reshape(N, 1, 128)
    stats = _stats(gp3, p3)

    Wp2b = jnp.concatenate([Wp2, bp2.reshape(1, C)], axis=0)
    return _attention(x_q.reshape(N, 1, C), gk.reshape(N, NS, C),
                      gv.reshape(N, NS, C), gp3, p3, stats, Wp1, bp1,
                      gamma, beta, Wp2b, bp2)


# speedup vs baseline: 3.3821x; 1.0174x over previous
import functools

import jax
import jax.numpy as jnp
from jax import lax
from jax.experimental import pallas as pl
from jax.experimental.pallas import tpu as pltpu
from jax.experimental.pallas import tpu_sc as plsc

N = 8192
C = 256
NS = 16


_QKV_R = 512


def _qkv_body(x_ref, wq_ref, bq_ref, wk_ref, bk_ref, wv_ref, bv_ref,
              q_ref, k_ref, v_ref):
    xb = x_ref[...]
    q_ref[...] = jnp.dot(xb, wq_ref[...],
                         preferred_element_type=jnp.float32) + bq_ref[...]
    k_ref[...] = jnp.dot(xb, wk_ref[...],
                         preferred_element_type=jnp.float32) + bk_ref[...]
    v_ref[...] = jnp.dot(xb, wv_ref[...],
                         preferred_element_type=jnp.float32) + bv_ref[...]


def _qkv(x, Wq, bq, Wk, bk, Wv, bv):
    grid = (N // _QKV_R,)
    row_spec = pl.BlockSpec((_QKV_R, C), lambda i: (i, 0))
    w_spec = pl.BlockSpec((C, C), lambda i: (0, 0))
    b_spec = pl.BlockSpec((1, C), lambda i: (0, 0))
    return pl.pallas_call(
        _qkv_body,
        grid=grid,
        in_specs=[row_spec, w_spec, b_spec, w_spec, b_spec, w_spec, b_spec],
        out_specs=[row_spec, row_spec, row_spec],
        out_shape=[jax.ShapeDtypeStruct((N, C), jnp.float32)] * 3,
    )(x, Wq, bq.reshape(1, C), Wk, bk.reshape(1, C), Wv, bv.reshape(1, C))


_KNN_R = 128


def _knn_body(pb_ref, pt_ref, idx_ref):
    pb = pb_ref[...]
    pt = pt_ref[...]
    sqb = jnp.sum(pb * pb, axis=1, keepdims=True)
    sqa = jnp.sum(pt * pt, axis=0, keepdims=True)
    d2 = sqb + sqa - 2.0 * jnp.dot(
        pb, pt, preferred_element_type=jnp.float32)
    iota = lax.broadcasted_iota(jnp.int32, d2.shape, 1)
    iota16 = lax.broadcasted_iota(jnp.int32, (_KNN_R, NS), 1)

    def step(t, carry):
        cur, acc = carry
        m = jnp.min(cur, axis=1, keepdims=True)
        cand = jnp.where(cur == m, iota, jnp.int32(2 ** 30))
        j = jnp.min(cand, axis=1, keepdims=True)
        cur = jnp.where(iota == j, jnp.float32(jnp.inf), cur)
        acc = jnp.where(iota16 == t, j, acc)
        return cur, acc

    _, acc = lax.fori_loop(
        0, NS, step, (d2, jnp.zeros((_KNN_R, NS), jnp.int32)))
    idx_ref[...] = acc


def _knn(p8, pT8):
    grid = (N // _KNN_R,)
    return pl.pallas_call(
        _knn_body,
        grid=grid,
        in_specs=[pl.BlockSpec((_KNN_R, 8), lambda i: (i, 0)),
                  pl.BlockSpec((8, N), lambda i: (0, 0))],
        out_specs=pl.BlockSpec((_KNN_R, NS), lambda i: (i, 0)),
        out_shape=jax.ShapeDtypeStruct((N, NS), jnp.int32),
    )(p8, pT8)


_G_CH = 128
_NW = 32
_B = N * NS
_PER_W = _B // _NW


def _gather_sc(idx_flat, xk, xv, px, py, pz):
    mesh = plsc.VectorSubcoreMesh(core_axis_name="c", subcore_axis_name="s")

    @functools.partial(
        pl.kernel,
        mesh=mesh,
        out_type=[
            jax.ShapeDtypeStruct((_B, C), jnp.float32),
            jax.ShapeDtypeStruct((_B, C), jnp.float32),
            jax.ShapeDtypeStruct((_B,), jnp.float32),
            jax.ShapeDtypeStruct((_B,), jnp.float32),
            jax.ShapeDtypeStruct((_B,), jnp.float32),
        ],
        scratch_types=[
            pltpu.VMEM((_G_CH,), jnp.int32),
            pltpu.VMEM((_G_CH, C), jnp.float32),
            pltpu.VMEM((_G_CH, C), jnp.float32),
            pltpu.VMEM_SHARED((N,), jnp.float32),
            pltpu.VMEM_SHARED((N,), jnp.float32),
            pltpu.VMEM_SHARED((N,), jnp.float32),
            pltpu.VMEM((_G_CH,), jnp.float32),
            pltpu.VMEM((_G_CH,), jnp.float32),
            pltpu.VMEM((_G_CH,), jnp.float32),
            pltpu.SemaphoreType.DMA,
            pltpu.SemaphoreType.DMA,
            pltpu.SemaphoreType.DMA,
            pltpu.SemaphoreType.DMA,
            pltpu.SemaphoreType.DMA,
        ],
    )
    def k(idx_hbm, xk_hbm, xv_hbm, px_hbm, py_hbm, pz_hbm,
          gk_hbm, gv_hbm, prx_hbm, pry_hbm, prz_hbm,
          idx_v, bk, bv, px_v, py_v, pz_v, bx, by, bz,
          s1, s2, s3, s4, s5):
        wid = lax.axis_index("s") * 2 + lax.axis_index("c")

        @pl.when(lax.axis_index("s") == 0)
        def _():
            pltpu.sync_copy(px_hbm, px_v)
            pltpu.sync_copy(py_hbm, py_v)
            pltpu.sync_copy(pz_hbm, pz_v)

        plsc.subcore_barrier()

        def body(ch, carry):
            base = wid * _PER_W + ch * _G_CH
            pltpu.sync_copy(idx_hbm.at[pl.ds(base, _G_CH)], idx_v)
            c1 = pltpu.async_copy(xk_hbm.at[idx_v], bk, s1)
            c2 = pltpu.async_copy(xv_hbm.at[idx_v], bv, s2)
            c3 = pltpu.async_copy(px_v.at[idx_v], bx, s3)
            c4 = pltpu.async_copy(py_v.at[idx_v], by, s4)
            c5 = pltpu.async_copy(pz_v.at[idx_v], bz, s5)
            c1.wait()
            c2.wait()
            c3.wait()
            c4.wait()
            c5.wait()
            pltpu.sync_copy(bk, gk_hbm.at[pl.ds(base, _G_CH)])
            pltpu.sync_copy(bv, gv_hbm.at[pl.ds(base, _G_CH)])
            pltpu.sync_copy(bx, prx_hbm.at[pl.ds(base, _G_CH)])
            pltpu.sync_copy(by, pry_hbm.at[pl.ds(base, _G_CH)])
            pltpu.sync_copy(bz, prz_hbm.at[pl.ds(base, _G_CH)])
            return carry

        lax.fori_loop(0, _PER_W // _G_CH, body, 0)

    return k(idx_flat, xk, xv, px, py, pz)


_ST_R = 1024


def _stats_body(px_ref, py_ref, pz_ref, p_ref, out_ref):
    pc = p_ref[...]
    d0 = px_ref[...] - pc[:, 0:1]
    d1 = py_ref[...] - pc[:, 1:2]
    d2 = pz_ref[...] - pc[:, 2:3]
    vals = [
        jnp.sum(d0), jnp.sum(d1), jnp.sum(d2),
        jnp.sum(d0 * d0), jnp.sum(d0 * d1), jnp.sum(d0 * d2),
        jnp.sum(d1 * d1), jnp.sum(d1 * d2), jnp.sum(d2 * d2),
    ]
    vec = jnp.concatenate(
        [v.reshape(1, 1) for v in vals] + [jnp.zeros((1, 7), jnp.float32)],
        axis=1)

    @pl.when(pl.program_id(0) == 0)
    def _():
        out_ref[...] = jnp.zeros_like(out_ref)

    out_ref[...] += vec


def _stats(prx, pry, prz, p8):
    grid = (N // _ST_R,)
    spec = pl.BlockSpec((_ST_R, NS), lambda i: (i, 0))
    return pl.pallas_call(
        _stats_body,
        grid=grid,
        in_specs=[spec, spec, spec,
                  pl.BlockSpec((_ST_R, 8), lambda i: (i, 0))],
        out_specs=pl.BlockSpec((1, 16), lambda i: (0, 0)),
        out_shape=jax.ShapeDtypeStruct((1, 16), jnp.float32),
    )(prx, pry, prz, p8)


_AT_R = 128


def _attn_body(xq_ref, gk_ref, gv_ref, px_ref, py_ref, pz_ref, p_ref,
               stats_ref, wp1_ref, bp1_ref, gamma_ref, beta_ref, wp2_ref,
               wp2t_ref, out_ref):
    pc = p_ref[...]
    prd = [px_ref[...] - pc[:, 0:1],
           py_ref[...] - pc[:, 1:2],
           pz_ref[...] - pc[:, 2:3]]

    M = float(N * NS)
    s1 = [stats_ref[0, i] for i in range(3)]
    s2 = {(0, 0): stats_ref[0, 3], (0, 1): stats_ref[0, 4],
          (0, 2): stats_ref[0, 5], (1, 1): stats_ref[0, 6],
          (1, 2): stats_ref[0, 7], (2, 2): stats_ref[0, 8]}

    rs = []
    for c in range(3):
        w = [wp1_ref[d, c] for d in range(3)]
        b = bp1_ref[c]
        h = prd[0] * w[0] + prd[1] * w[1] + prd[2] * w[2] + b
        sw = s1[0] * w[0] + s1[1] * w[1] + s1[2] * w[2]
        mean = sw / M + b
        ex2 = (w[0] * w[0] * s2[(0, 0)] + w[1] * w[1] * s2[(1, 1)]
               + w[2] * w[2] * s2[(2, 2)]
               + 2.0 * (w[0] * w[1] * s2[(0, 1)]
                        + w[0] * w[2] * s2[(0, 2)]
                        + w[1] * w[2] * s2[(1, 2)])) / M \
            + 2.0 * b * sw / M + b * b
        var = ex2 - mean * mean
        inv = gamma_ref[c] / jnp.sqrt(var + 1e-5)
        hn = (h - mean) * inv + beta_ref[c]
        rs.append(jnp.maximum(hn, 0.0))

    xq3 = xq_ref[...]
    xq2 = xq3.reshape(_AT_R, C)
    qv = jnp.dot(xq2, wp2t_ref[...],
                 preferred_element_type=jnp.float32)
    attn = jnp.sum(xq3 * gk_ref[...], axis=2)
    attn = attn + qv[:, 3:4]
    for c in range(3):
        attn = attn + rs[c] * qv[:, c:c + 1]
    attn = attn * (1.0 / 16.0)
    attn = attn - jnp.max(attn, axis=1, keepdims=True)
    e = jnp.exp(attn)
    wgt = e / jnp.sum(e, axis=1, keepdims=True)

    wgt3 = wgt.reshape(_AT_R, NS, 1)
    acc = jnp.sum(gv_ref[...] * wgt3, axis=1)
    acc = acc + wp2_ref[3, :].reshape(1, C)
    for c in range(3):
        s3 = jnp.sum(wgt * rs[c], axis=1, keepdims=True)
        acc = acc + s3 * wp2_ref[c, :].reshape(1, C)
    out_ref[...] = acc


def _attention(xq3, gk3, gv3, prx, pry, prz, p8, stats, Wp1, bp1, gamma,
               beta, Wp2b, Wp2t):
    grid = (N // _AT_R,)
    smem = functools.partial(pl.BlockSpec, memory_space=pltpu.SMEM)
    pspec = pl.BlockSpec((_AT_R, NS), lambda i: (i, 0))
    return pl.pallas_call(
        _attn_body,
        grid=grid,
        in_specs=[
            pl.BlockSpec((_AT_R, 1, C), lambda i: (i, 0, 0)),
            pl.BlockSpec((_AT_R, NS, C), lambda i: (i, 0, 0)),
            pl.BlockSpec((_AT_R, NS, C), lambda i: (i, 0, 0)),
            pspec,
            pspec,
            pspec,
            pl.BlockSpec((_AT_R, 8), lambda i: (i, 0)),
            smem(),
            smem(),
            smem(),
            smem(),
            smem(),
            pl.BlockSpec((4, C), lambda i: (0, 0)),
            pl.BlockSpec((C, 128), lambda i: (0, 0)),
        ],
        out_specs=pl.BlockSpec((_AT_R, C), lambda i: (i, 0)),
        out_shape=jax.ShapeDtypeStruct((N, C), jnp.float32),
    )(xq3, gk3, gv3, prx, pry, prz, p8, stats, Wp1, bp1, gamma, beta,
      Wp2b, Wp2t)


def kernel(p, x, o, Wq, bq, Wk, bk, Wv, bv, Wp1, bp1, gamma, beta, Wp2,
           bp2):
    del o
    x_q, x_k, x_v = _qkv(x, Wq, bq, Wk, bk, Wv, bv)

    p8 = jnp.pad(p, ((0, 0), (0, 5)))
    pT8 = p8.T
    idx = _knn(p8, pT8)

    px, py, pz = p[:, 0], p[:, 1], p[:, 2]
    gk, gv, prx, pry, prz = _gather_sc(idx.reshape(_B), x_k, x_v,
                                       px, py, pz)
    prx = prx.reshape(N, NS)
    pry = pry.reshape(N, NS)
    prz = prz.reshape(N, NS)
    stats = _stats(prx, pry, prz, p8)

    Wp2b = jnp.concatenate([Wp2, bp2.reshape(1, C)], axis=0)
    Wp2t = jnp.pad(Wp2b.T, ((0, 0), (0, 124)))
    return _attention(x_q.reshape(N, 1, C), gk.reshape(N, NS, C),
                      gv.reshape(N, NS, C), prx, pry, prz, p8, stats,
                      Wp1, bp1, gamma, beta, Wp2b, Wp2t)


# speedup vs baseline: 3.5473x; 1.0488x over previous
import functools

import jax
import jax.numpy as jnp
from jax import lax
from jax.experimental import pallas as pl
from jax.experimental.pallas import tpu as pltpu
from jax.experimental.pallas import tpu_sc as plsc

N = 8192
C = 256
NS = 16


_QKV_R = 512


def _qkv_body(x_ref, wq_ref, bq_ref, wk_ref, bk_ref, wv_ref, bv_ref,
              q_ref, k_ref, v_ref):
    xb = x_ref[...]
    q_ref[...] = jnp.dot(xb, wq_ref[...],
                         preferred_element_type=jnp.float32) + bq_ref[...]
    k_ref[...] = jnp.dot(xb, wk_ref[...],
                         preferred_element_type=jnp.float32) + bk_ref[...]
    v_ref[...] = jnp.dot(xb, wv_ref[...],
                         preferred_element_type=jnp.float32) + bv_ref[...]


def _qkv(x, Wq, bq, Wk, bk, Wv, bv):
    grid = (N // _QKV_R,)
    row_spec = pl.BlockSpec((_QKV_R, C), lambda i: (i, 0))
    w_spec = pl.BlockSpec((C, C), lambda i: (0, 0))
    b_spec = pl.BlockSpec((1, C), lambda i: (0, 0))
    return pl.pallas_call(
        _qkv_body,
        grid=grid,
        in_specs=[row_spec, w_spec, b_spec, w_spec, b_spec, w_spec, b_spec],
        out_specs=[row_spec, row_spec, row_spec],
        out_shape=[jax.ShapeDtypeStruct((N, C), jnp.float32)] * 3,
    )(x, Wq, bq.reshape(1, C), Wk, bk.reshape(1, C), Wv, bv.reshape(1, C))


_KNN_R = 256


def _knn_body(pb_ref, pt_ref, idx_ref):
    pb = pb_ref[...]
    pt = pt_ref[...]
    sqb = jnp.sum(pb * pb, axis=1, keepdims=True)
    sqa = jnp.sum(pt * pt, axis=0, keepdims=True)
    d2 = sqb + sqa - 2.0 * jnp.dot(
        pb, pt, preferred_element_type=jnp.float32)
    iota = lax.broadcasted_iota(jnp.int32, d2.shape, 1)
    iota16 = lax.broadcasted_iota(jnp.int32, (_KNN_R, NS), 1)

    def step(t, carry):
        cur, acc = carry
        m = jnp.min(cur, axis=1, keepdims=True)
        cand = jnp.where(cur == m, iota, jnp.int32(2 ** 30))
        j = jnp.min(cand, axis=1, keepdims=True)
        cur = jnp.where(iota == j, jnp.float32(jnp.inf), cur)
        acc = jnp.where(iota16 == t, j, acc)
        return cur, acc

    _, acc = lax.fori_loop(
        0, NS, step, (d2, jnp.zeros((_KNN_R, NS), jnp.int32)))
    idx_ref[...] = acc


def _knn(p8, pT8):
    grid = (N // _KNN_R,)
    return pl.pallas_call(
        _knn_body,
        grid=grid,
        in_specs=[pl.BlockSpec((_KNN_R, 8), lambda i: (i, 0)),
                  pl.BlockSpec((8, N), lambda i: (0, 0))],
        out_specs=pl.BlockSpec((_KNN_R, NS), lambda i: (i, 0)),
        out_shape=jax.ShapeDtypeStruct((N, NS), jnp.int32),
    )(p8, pT8)


_G_CH = 128
_NW = 32
_B = N * NS
_PER_W = _B // _NW


def _gather_sc(idx_flat, xk, xv, px, py, pz):
    mesh = plsc.VectorSubcoreMesh(core_axis_name="c", subcore_axis_name="s")

    @functools.partial(
        pl.kernel,
        mesh=mesh,
        out_type=[
            jax.ShapeDtypeStruct((_B, C), jnp.float32),
            jax.ShapeDtypeStruct((_B, C), jnp.float32),
            jax.ShapeDtypeStruct((_B,), jnp.float32),
            jax.ShapeDtypeStruct((_B,), jnp.float32),
            jax.ShapeDtypeStruct((_B,), jnp.float32),
        ],
        scratch_types=[
            pltpu.VMEM((_G_CH,), jnp.int32),
            pltpu.VMEM((_G_CH, C), jnp.float32),
            pltpu.VMEM((_G_CH, C), jnp.float32),
            pltpu.VMEM_SHARED((N,), jnp.float32),
            pltpu.VMEM_SHARED((N,), jnp.float32),
            pltpu.VMEM_SHARED((N,), jnp.float32),
            pltpu.VMEM((_G_CH,), jnp.float32),
            pltpu.VMEM((_G_CH,), jnp.float32),
            pltpu.VMEM((_G_CH,), jnp.float32),
            pltpu.SemaphoreType.DMA,
            pltpu.SemaphoreType.DMA,
            pltpu.SemaphoreType.DMA,
            pltpu.SemaphoreType.DMA,
            pltpu.SemaphoreType.DMA,
        ],
    )
    def k(idx_hbm, xk_hbm, xv_hbm, px_hbm, py_hbm, pz_hbm,
          gk_hbm, gv_hbm, prx_hbm, pry_hbm, prz_hbm,
          idx_v, bk, bv, px_v, py_v, pz_v, bx, by, bz,
          s1, s2, s3, s4, s5):
        wid = lax.axis_index("s") * 2 + lax.axis_index("c")

        @pl.when(lax.axis_index("s") == 0)
        def _():
            pltpu.sync_copy(px_hbm, px_v)
            pltpu.sync_copy(py_hbm, py_v)
            pltpu.sync_copy(pz_hbm, pz_v)

        plsc.subcore_barrier()

        def body(ch, carry):
            base = wid * _PER_W + ch * _G_CH
            pltpu.sync_copy(idx_hbm.at[pl.ds(base, _G_CH)], idx_v)
            c1 = pltpu.async_copy(xk_hbm.at[idx_v], bk, s1)
            c2 = pltpu.async_copy(xv_hbm.at[idx_v], bv, s2)
            c3 = pltpu.async_copy(px_v.at[idx_v], bx, s3)
            c4 = pltpu.async_copy(py_v.at[idx_v], by, s4)
            c5 = pltpu.async_copy(pz_v.at[idx_v], bz, s5)
            c1.wait()
            c2.wait()
            c3.wait()
            c4.wait()
            c5.wait()
            pltpu.sync_copy(bk, gk_hbm.at[pl.ds(base, _G_CH)])
            pltpu.sync_copy(bv, gv_hbm.at[pl.ds(base, _G_CH)])
            pltpu.sync_copy(bx, prx_hbm.at[pl.ds(base, _G_CH)])
            pltpu.sync_copy(by, pry_hbm.at[pl.ds(base, _G_CH)])
            pltpu.sync_copy(bz, prz_hbm.at[pl.ds(base, _G_CH)])
            return carry

        lax.fori_loop(0, _PER_W // _G_CH, body, 0)

    return k(idx_flat, xk, xv, px, py, pz)


_ST_R = 1024


def _stats_body(px_ref, py_ref, pz_ref, p_ref, out_ref):
    pc = p_ref[...]
    d0 = px_ref[...] - pc[:, 0:1]
    d1 = py_ref[...] - pc[:, 1:2]
    d2 = pz_ref[...] - pc[:, 2:3]
    vals = [
        jnp.sum(d0), jnp.sum(d1), jnp.sum(d2),
        jnp.sum(d0 * d0), jnp.sum(d0 * d1), jnp.sum(d0 * d2),
        jnp.sum(d1 * d1), jnp.sum(d1 * d2), jnp.sum(d2 * d2),
    ]
    vec = jnp.concatenate(
        [v.reshape(1, 1) for v in vals] + [jnp.zeros((1, 7), jnp.float32)],
        axis=1)

    @pl.when(pl.program_id(0) == 0)
    def _():
        out_ref[...] = jnp.zeros_like(out_ref)

    out_ref[...] += vec


def _stats(prx, pry, prz, p8):
    grid = (N // _ST_R,)
    spec = pl.BlockSpec((_ST_R, NS), lambda i: (i, 0))
    return pl.pallas_call(
        _stats_body,
        grid=grid,
        in_specs=[spec, spec, spec,
                  pl.BlockSpec((_ST_R, 8), lambda i: (i, 0))],
        out_specs=pl.BlockSpec((1, 16), lambda i: (0, 0)),
        out_shape=jax.ShapeDtypeStruct((1, 16), jnp.float32),
    )(prx, pry, prz, p8)


_AT_R = 256


def _attn_body(xq_ref, gk_ref, gv_ref, px_ref, py_ref, pz_ref, p_ref,
               stats_ref, wp1_ref, bp1_ref, gamma_ref, beta_ref, wp2_ref,
               wp2t_ref, out_ref):
    pc = p_ref[...]
    prd = [px_ref[...] - pc[:, 0:1],
           py_ref[...] - pc[:, 1:2],
           pz_ref[...] - pc[:, 2:3]]

    M = float(N * NS)
    s1 = [stats_ref[0, i] for i in range(3)]
    s2 = {(0, 0): stats_ref[0, 3], (0, 1): stats_ref[0, 4],
          (0, 2): stats_ref[0, 5], (1, 1): stats_ref[0, 6],
          (1, 2): stats_ref[0, 7], (2, 2): stats_ref[0, 8]}

    rs = []
    for c in range(3):
        w = [wp1_ref[d, c] for d in range(3)]
        b = bp1_ref[c]
        h = prd[0] * w[0] + prd[1] * w[1] + prd[2] * w[2] + b
        sw = s1[0] * w[0] + s1[1] * w[1] + s1[2] * w[2]
        mean = sw / M + b
        ex2 = (w[0] * w[0] * s2[(0, 0)] + w[1] * w[1] * s2[(1, 1)]
               + w[2] * w[2] * s2[(2, 2)]
               + 2.0 * (w[0] * w[1] * s2[(0, 1)]
                        + w[0] * w[2] * s2[(0, 2)]
                        + w[1] * w[2] * s2[(1, 2)])) / M \
            + 2.0 * b * sw / M + b * b
        var = ex2 - mean * mean
        inv = gamma_ref[c] / jnp.sqrt(var + 1e-5)
        hn = (h - mean) * inv + beta_ref[c]
        rs.append(jnp.maximum(hn, 0.0))

    xq3 = xq_ref[...]
    xq2 = xq3.reshape(_AT_R, C)
    qv = jnp.dot(xq2, wp2t_ref[...],
                 preferred_element_type=jnp.float32)
    attn = jnp.sum(xq3 * gk_ref[...], axis=2)
    attn = attn + qv[:, 3:4]
    for c in range(3):
        attn = attn + rs[c] * qv[:, c:c + 1]
    attn = attn * (1.0 / 16.0)
    attn = attn - jnp.max(attn, axis=1, keepdims=True)
    e = jnp.exp(attn)
    wgt = e / jnp.sum(e, axis=1, keepdims=True)

    wgt3 = wgt.reshape(_AT_R, NS, 1)
    acc = jnp.sum(gv_ref[...] * wgt3, axis=1)
    acc = acc + wp2_ref[3, :].reshape(1, C)
    for c in range(3):
        s3 = jnp.sum(wgt * rs[c], axis=1, keepdims=True)
        acc = acc + s3 * wp2_ref[c, :].reshape(1, C)
    out_ref[...] = acc


def _attention(xq3, gk3, gv3, prx, pry, prz, p8, stats, Wp1, bp1, gamma,
               beta, Wp2b, Wp2t):
    grid = (N // _AT_R,)
    smem = functools.partial(pl.BlockSpec, memory_space=pltpu.SMEM)
    pspec = pl.BlockSpec((_AT_R, NS), lambda i: (i, 0))
    return pl.pallas_call(
        _attn_body,
        grid=grid,
        in_specs=[
            pl.BlockSpec((_AT_R, 1, C), lambda i: (i, 0, 0)),
            pl.BlockSpec((_AT_R, NS, C), lambda i: (i, 0, 0)),
            pl.BlockSpec((_AT_R, NS, C), lambda i: (i, 0, 0)),
            pspec,
            pspec,
            pspec,
            pl.BlockSpec((_AT_R, 8), lambda i: (i, 0)),
            smem(),
            smem(),
            smem(),
            smem(),
            smem(),
            pl.BlockSpec((4, C), lambda i: (0, 0)),
            pl.BlockSpec((C, 128), lambda i: (0, 0)),
        ],
        out_specs=pl.BlockSpec((_AT_R, C), lambda i: (i, 0)),
        out_shape=jax.ShapeDtypeStruct((N, C), jnp.float32),
    )(xq3, gk3, gv3, prx, pry, prz, p8, stats, Wp1, bp1, gamma, beta,
      Wp2b, Wp2t)


def kernel(p, x, o, Wq, bq, Wk, bk, Wv, bv, Wp1, bp1, gamma, beta, Wp2,
           bp2):
    del o
    x_q, x_k, x_v = _qkv(x, Wq, bq, Wk, bk, Wv, bv)

    p8 = jnp.pad(p, ((0, 0), (0, 5)))
    pT8 = p8.T
    idx = _knn(p8, pT8)

    px, py, pz = p[:, 0], p[:, 1], p[:, 2]
    gk, gv, prx, pry, prz = _gather_sc(idx.reshape(_B), x_k, x_v,
                                       px, py, pz)
    prx = prx.reshape(N, NS)
    pry = pry.reshape(N, NS)
    prz = prz.reshape(N, NS)
    stats = _stats(prx, pry, prz, p8)

    Wp2b = jnp.concatenate([Wp2, bp2.reshape(1, C)], axis=0)
    Wp2t = jnp.pad(Wp2b.T, ((0, 0), (0, 124)))
    return _attention(x_q.reshape(N, 1, C), gk.reshape(N, NS, C),
                      gv.reshape(N, NS, C), prx, pry, prz, p8, stats,
                      Wp1, bp1, gamma, beta, Wp2b, Wp2t)
